# baseline (speedup 1.0000x reference)
import functools

import jax
import jax.numpy as jnp
from jax import lax
from jax.experimental import pallas as pl
from jax.experimental.pallas import tpu as pltpu
from jax.experimental.pallas import tpu_sc as plsc

B = 4096
CTX = 20
D = 64
V = 100000

NC = 2
NS = 16
NW = NC * NS
D_PER_W = D // NW
CHUNKB = 512
N_CHUNKS = B // CHUNKB

_sc_mesh = plsc.VectorSubcoreMesh(core_axis_name="c", subcore_axis_name="s")


@functools.partial(
    pl.kernel,
    out_type=jax.ShapeDtypeStruct((D + 1, B), jnp.float32),
    mesh=_sc_mesh,
    scratch_types=[
        pltpu.VMEM((2, CTX, CHUNKB), jnp.int32),
        pltpu.VMEM((V,), jnp.float32),
        pltpu.VMEM((B,), jnp.float32),
        pltpu.VMEM((1, B // NW), jnp.float32),
        pltpu.SemaphoreType.DMA,
        pltpu.SemaphoreType.DMA,
    ],
    compiler_params=pltpu.CompilerParams(
        use_tc_tiling_on_sc=True, needs_layout_passes=False),
)
def _pool_sc(idx_hbm, tab_hbm, out_hbm, idx_v, row_v, out_v, ones_v, sem0, sem1):
    wid = lax.axis_index("s") * NC + lax.axis_index("c")
    sems = (sem0, sem1)
    for k in range(B // NW // 16):
        ones_v[0, pl.ds(k * 16, 16)] = jnp.ones((16,), jnp.float32)
    pltpu.sync_copy(
        ones_v, out_hbm.at[pl.ds(D, 1), pl.ds(wid * (B // NW), B // NW)])
    for f in range(D_PER_W):
        d = wid * D_PER_W + f
        row_cp = pltpu.async_copy(tab_hbm.at[d], row_v, sem1)
        pltpu.async_copy(
            idx_hbm.at[:, pl.ds(0, CHUNKB)], idx_v.at[0], sem0).wait()
        row_cp.wait()
        for c in range(N_CHUNKS):
            buf = c % 2
            if c + 1 < N_CHUNKS:
                nxt = pltpu.async_copy(
                    idx_hbm.at[:, pl.ds((c + 1) * CHUNKB, CHUNKB)],
                    idx_v.at[(c + 1) % 2], sems[(c + 1) % 2])

            @plsc.parallel_loop(0, CHUNKB // 16)
            def vec_body(bv, buf=buf, c=c):
                sl = pl.ds(bv * 16, 16)
                acc = plsc.load_gather(row_v, [idx_v[buf, 0, sl]])
                for l in range(1, CTX):
                    acc = acc + plsc.load_gather(row_v, [idx_v[buf, l, sl]])
                out_v[pl.ds(c * CHUNKB + bv * 16, 16)] = acc * (1.0 / CTX)

            if c + 1 < N_CHUNKS:
                nxt.wait()
        pltpu.sync_copy(out_v, out_hbm.at[d])


VB = 1024

_TN = (((0,), (0,)), ((), ()))


def _mm_body(w_ref, xt_ref, o_ref):
    o_ref[...] = jax.lax.dot_general(w_ref[...], xt_ref[...], _TN,
                                     preferred_element_type=jnp.float32)


_matmul_t = pl.pallas_call(
    _mm_body,
    grid=(pl.cdiv(V, VB),),
    in_specs=[
        pl.BlockSpec((D + 1, VB), lambda i: (0, i)),
        pl.BlockSpec((D + 1, B), lambda i: (0, 0)),
    ],
    out_specs=pl.BlockSpec((VB, B), lambda i: (i, 0)),
    out_shape=jax.ShapeDtypeStruct((V, B), jnp.float32),
    compiler_params=pltpu.CompilerParams(
        dimension_semantics=("arbitrary",), vmem_limit_bytes=128*1024*1024),
)


def kernel(inputs, embedding_table, fc_w, fc_b):
    idx_t = inputs.T.astype(jnp.int32)
    table_t = embedding_table.T
    pooled_t = _pool_sc(idx_t, table_t)
    w_aug = jnp.concatenate([fc_w, fc_b.reshape(1, V)], axis=0)
    logits_t = _matmul_t(w_aug, pooled_t)
    return logits_t.T
